# baseline (speedup 1.0000x reference)
import functools

import jax
import jax.numpy as jnp
from jax import lax
from jax.experimental import pallas as pl
from jax.experimental.pallas import tpu as pltpu
from jax.experimental.pallas import tpu_sc as plsc

_N = 10000
_D = 128
_E = 320000

_NPAD = 10240
_BR = 512
_GRID = (_N + _BR - 1) // _BR

_NC = 2
_NS = 16
_NW = _NC * _NS

_C = 128
_TCH = _E // _C
_BASE_CH = _TCH // _NW
_REM_CH = _TCH % _NW
_ROWS_PER_TILE = _NPAD // _NS


@functools.partial(
    pl.kernel,
    mesh=plsc.VectorSubcoreMesh(core_axis_name="c", subcore_axis_name="s"),
    out_type=jax.ShapeDtypeStruct((_NC, _NPAD, _D), jnp.float32),
    scratch_types=[
        pltpu.VMEM((2, _C), jnp.int32),
        pltpu.VMEM((2, _C), jnp.int32),
        pltpu.VMEM((2, _C), jnp.float32),
        pltpu.VMEM((2, _C, _D), jnp.float32),
        pltpu.VMEM_SHARED((_NPAD, _D), jnp.float32),
        pltpu.SemaphoreType.DMA((2,)),
        pltpu.SemaphoreType.DMA((2,)),
    ],
)
def _sc_segsum(a_hbm, ei_hbm, ew_hbm, out_hbm, srcv, dstv, ewv, rows, acc, isem, gsem):
    cid = lax.axis_index("c")
    sid = lax.axis_index("s")
    wid = sid * _NC + cid

    def _zrow(r, carry):
        for g in range(_D // 16):
            rows[0, r, pl.ds(g * 16, 16)] = jnp.zeros((16,), jnp.float32)
        return carry

    lax.fori_loop(0, _C, _zrow, 0)
    r0 = sid * _ROWS_PER_TILE
    for b in range(_ROWS_PER_TILE // _C):
        pltpu.sync_copy(rows.at[0], acc.at[pl.ds(r0 + b * _C, _C)])
    plsc.subcore_barrier()

    nch = _BASE_CH + (wid < _REM_CH).astype(jnp.int32)

    def _idx_copies(k, slot):
        base = (wid + k * _NW) * _C
        return (
            pltpu.make_async_copy(ei_hbm.at[0, pl.ds(base, _C)], srcv.at[slot], isem.at[slot]),
            pltpu.make_async_copy(ei_hbm.at[1, pl.ds(base, _C)], dstv.at[slot], isem.at[slot]),
            pltpu.make_async_copy(ew_hbm.at[pl.ds(base, _C)], ewv.at[slot], isem.at[slot]),
        )

    def _idx_start(k, slot):
        for c in _idx_copies(k, slot):
            c.start()

    def _idx_wait(k, slot):
        for c in _idx_copies(k, slot):
            c.wait()

    def _gather_start(b):
        pltpu.make_async_copy(a_hbm.at[srcv.at[b]], rows.at[b], gsem.at[b]).start()

    def _gather_wait(b):
        pltpu.make_async_copy(a_hbm.at[srcv.at[b]], rows.at[b], gsem.at[b]).wait()

    _idx_start(0, 0)
    _idx_wait(0, 0)
    _gather_start(0)
    _idx_start(1, 1)

    def _process(k, b):
        @pl.when(k + 1 < nch)
        def _pref():
            _idx_wait(k + 1, 1 - b)
            _gather_start(1 - b)

        _gather_wait(b)

        def _escale(g, c2):
            w16 = ewv[b, pl.ds(g * 16, 16)]
            for jj in range(16):
                wj = w16[jj]
                e = g * 16 + jj
                for gg in range(_D // 16):
                    rows[b, e, pl.ds(gg * 16, 16)] = rows[b, e, pl.ds(gg * 16, 16)] * wj
            return c2

        lax.fori_loop(0, _C // 16, _escale, 0)
        pltpu.sync_copy(rows.at[b], acc.at[dstv.at[b]], add=True)

        @pl.when(k + 2 < nch)
        def _prefidx():
            _idx_start(k + 2, b)

    def _pair(kk, carry):
        k0 = 2 * kk

        @pl.when(k0 < nch)
        def _p0():
            _process(k0, 0)

        @pl.when(k0 + 1 < nch)
        def _p1():
            _process(k0 + 1, 1)

        return carry

    lax.fori_loop(0, (_BASE_CH + 2) // 2, _pair, 0)
    plsc.subcore_barrier()

    for b in range(_ROWS_PER_TILE // _C):
        r = r0 + b * _C
        pltpu.sync_copy(acc.at[pl.ds(r, _C)], out_hbm.at[cid, pl.ds(r, _C)])


def _tc_first_body(x_ref, wr_ref, b_ref, wo_ref, a_ref, r_ref):
    h = x_ref[...]
    a_ref[...] = jnp.dot(h, wr_ref[...], preferred_element_type=jnp.float32)
    r_ref[...] = jnp.dot(h, wo_ref[...], preferred_element_type=jnp.float32) + b_ref[...]


def _tc_mid_body(p_ref, rp_ref, wr_ref, b_ref, wo_ref, a_ref, r_ref):
    h = jnp.maximum(p_ref[0] + p_ref[1] + rp_ref[...], 0.0)
    a_ref[...] = jnp.dot(h, wr_ref[...], preferred_element_type=jnp.float32)
    r_ref[...] = jnp.dot(h, wo_ref[...], preferred_element_type=jnp.float32) + b_ref[...]


def _tc_last_body(p_ref, rp_ref, o_ref):
    o_ref[...] = p_ref[0] + p_ref[1] + rp_ref[...]


_W_SPEC = pl.BlockSpec((_D, _D), lambda i: (0, 0))
_B_SPEC = pl.BlockSpec((1, _D), lambda i: (0, 0))
_ROW_SPEC = pl.BlockSpec((_BR, _D), lambda i: (i, 0))
_P_SPEC = pl.BlockSpec((_NC, _BR, _D), lambda i: (0, i, 0))


def _mm_first(x, wr, b, wo):
    return pl.pallas_call(
        _tc_first_body,
        grid=(_GRID,),
        in_specs=[_ROW_SPEC, _W_SPEC, _B_SPEC, _W_SPEC],
        out_specs=[_ROW_SPEC, _ROW_SPEC],
        out_shape=[jax.ShapeDtypeStruct((_N, _D), jnp.float32)] * 2,
    )(x, wr, b.reshape(1, _D), wo)


def _mm_mid(p, rp, wr, b, wo):
    return pl.pallas_call(
        _tc_mid_body,
        grid=(_GRID,),
        in_specs=[_P_SPEC, _ROW_SPEC, _W_SPEC, _B_SPEC, _W_SPEC],
        out_specs=[_ROW_SPEC, _ROW_SPEC],
        out_shape=[jax.ShapeDtypeStruct((_N, _D), jnp.float32)] * 2,
    )(p, rp, wr, b.reshape(1, _D), wo)


def _mm_last(p, rp):
    return pl.pallas_call(
        _tc_last_body,
        grid=(_GRID,),
        in_specs=[_P_SPEC, _ROW_SPEC],
        out_specs=_ROW_SPEC,
        out_shape=jax.ShapeDtypeStruct((_N, _D), jnp.float32),
    )(p, rp)


def kernel(x, edge_index, edge_weight,
           W_rel_0, b_rel_0, W_root_0,
           W_rel_1, b_rel_1, W_root_1,
           W_rel_2, b_rel_2, W_root_2):
    a, r = _mm_first(x, W_rel_0, b_rel_0, W_root_0)
    p = _sc_segsum(a, edge_index, edge_weight)
    a, r = _mm_mid(p, r, W_rel_1, b_rel_1, W_root_1)
    p = _sc_segsum(a, edge_index, edge_weight)
    a, r = _mm_mid(p, r, W_rel_2, b_rel_2, W_root_2)
    p = _sc_segsum(a, edge_index, edge_weight)
    return _mm_last(p, r)


# speedup vs baseline: 1.1902x; 1.1902x over previous
import functools

import jax
import jax.numpy as jnp
from jax import lax
from jax.experimental import pallas as pl
from jax.experimental.pallas import tpu as pltpu
from jax.experimental.pallas import tpu_sc as plsc

_N = 10000
_D = 128
_E = 320000

_NPAD = 10240
_BR = 512
_GRID = (_N + _BR - 1) // _BR

_NC = 2
_NS = 16
_NW = _NC * _NS

_C = 128
_TCH = _E // _C
_BASE_CH = _TCH // _NW
_REM_CH = _TCH % _NW
_ROWS_PER_TILE = _NPAD // _NS


@functools.partial(
    pl.kernel,
    mesh=plsc.VectorSubcoreMesh(core_axis_name="c", subcore_axis_name="s"),
    out_type=jax.ShapeDtypeStruct((_NC, _NPAD, _D), jnp.float32),
    scratch_types=[
        pltpu.VMEM((2, _C), jnp.int32),
        pltpu.VMEM((2, _C), jnp.int32),
        pltpu.VMEM((2, _C), jnp.float32),
        pltpu.VMEM((2, _C, _D), jnp.float32),
        pltpu.VMEM_SHARED((_NPAD, _D), jnp.float32),
        pltpu.SemaphoreType.DMA((2,)),
        pltpu.SemaphoreType.DMA((2,)),
        pltpu.SemaphoreType.DMA((2,)),
    ],
)
def _sc_segsum(a_hbm, ei_hbm, ew_hbm, out_hbm, srcv, dstv, ewv, rows, acc, isem, gsem, ssem):
    cid = lax.axis_index("c")
    sid = lax.axis_index("s")
    wid = sid * _NC + cid

    def _zrow(r, carry):
        for g in range(_D // 16):
            rows[0, r, pl.ds(g * 16, 16)] = jnp.zeros((16,), jnp.float32)
        return carry

    lax.fori_loop(0, _C, _zrow, 0)
    r0 = sid * _ROWS_PER_TILE
    for b in range(_ROWS_PER_TILE // _C):
        pltpu.sync_copy(rows.at[0], acc.at[pl.ds(r0 + b * _C, _C)])
    plsc.subcore_barrier()

    nch = _BASE_CH + (wid < _REM_CH).astype(jnp.int32)

    def _idx_copies(k, slot):
        base = (wid + k * _NW) * _C
        return (
            pltpu.make_async_copy(ei_hbm.at[0, pl.ds(base, _C)], srcv.at[slot], isem.at[slot]),
            pltpu.make_async_copy(ei_hbm.at[1, pl.ds(base, _C)], dstv.at[slot], isem.at[slot]),
            pltpu.make_async_copy(ew_hbm.at[pl.ds(base, _C)], ewv.at[slot], isem.at[slot]),
        )

    def _idx_start(k, slot):
        for c in _idx_copies(k, slot):
            c.start()

    def _idx_wait(k, slot):
        for c in _idx_copies(k, slot):
            c.wait()

    def _gather_start(b):
        pltpu.make_async_copy(a_hbm.at[srcv.at[b]], rows.at[b], gsem.at[b]).start()

    def _gather_wait(b):
        pltpu.make_async_copy(a_hbm.at[srcv.at[b]], rows.at[b], gsem.at[b]).wait()

    _idx_start(0, 0)
    _idx_wait(0, 0)
    _gather_start(0)
    _idx_start(1, 1)

    def _scat_copy(b):
        return pltpu.make_async_copy(rows.at[b], acc.at[dstv.at[b]], ssem.at[b])

    def _process(k, b):
        @pl.when(k + 1 < nch)
        def _pref():
            _idx_wait(k + 1, 1 - b)

            @pl.when(k >= 1)
            def _drain_prev():
                _scat_copy(1 - b).wait()

            _gather_start(1 - b)

        _gather_wait(b)

        def _escale(g, c2):
            w16 = ewv[b, pl.ds(g * 16, 16)]
            for jj in range(16):
                wj = w16[jj]
                e = g * 16 + jj
                for gg in range(_D // 16):
                    rows[b, e, pl.ds(gg * 16, 16)] = rows[b, e, pl.ds(gg * 16, 16)] * wj
            return c2

        lax.fori_loop(0, _C // 16, _escale, 0)
        _scat_copy(b).start(add=True)

        @pl.when(k + 2 < nch)
        def _prefidx():
            _idx_start(k + 2, b)

    def _pair(kk, carry):
        k0 = 2 * kk

        @pl.when(k0 < nch)
        def _p0():
            _process(k0, 0)

        @pl.when(k0 + 1 < nch)
        def _p1():
            _process(k0 + 1, 1)

        return carry

    lax.fori_loop(0, (_BASE_CH + 2) // 2, _pair, 0)
    _scat_copy(0).wait()
    _scat_copy(1).wait()
    plsc.subcore_barrier()

    for b in range(_ROWS_PER_TILE // _C):
        r = r0 + b * _C
        pltpu.sync_copy(acc.at[pl.ds(r, _C)], out_hbm.at[cid, pl.ds(r, _C)])


def _tc_first_body(x_ref, wr_ref, b_ref, wo_ref, a_ref, r_ref):
    h = x_ref[...]
    a_ref[...] = jnp.dot(h, wr_ref[...], preferred_element_type=jnp.float32)
    r_ref[...] = jnp.dot(h, wo_ref[...], preferred_element_type=jnp.float32) + b_ref[...]


def _tc_mid_body(p_ref, rp_ref, wr_ref, b_ref, wo_ref, a_ref, r_ref):
    h = jnp.maximum(p_ref[0] + p_ref[1] + rp_ref[...], 0.0)
    a_ref[...] = jnp.dot(h, wr_ref[...], preferred_element_type=jnp.float32)
    r_ref[...] = jnp.dot(h, wo_ref[...], preferred_element_type=jnp.float32) + b_ref[...]


def _tc_last_body(p_ref, rp_ref, o_ref):
    o_ref[...] = p_ref[0] + p_ref[1] + rp_ref[...]


_W_SPEC = pl.BlockSpec((_D, _D), lambda i: (0, 0))
_B_SPEC = pl.BlockSpec((1, _D), lambda i: (0, 0))
_ROW_SPEC = pl.BlockSpec((_BR, _D), lambda i: (i, 0))
_P_SPEC = pl.BlockSpec((_NC, _BR, _D), lambda i: (0, i, 0))


def _mm_first(x, wr, b, wo):
    return pl.pallas_call(
        _tc_first_body,
        grid=(_GRID,),
        in_specs=[_ROW_SPEC, _W_SPEC, _B_SPEC, _W_SPEC],
        out_specs=[_ROW_SPEC, _ROW_SPEC],
        out_shape=[jax.ShapeDtypeStruct((_N, _D), jnp.float32)] * 2,
    )(x, wr, b.reshape(1, _D), wo)


def _mm_mid(p, rp, wr, b, wo):
    return pl.pallas_call(
        _tc_mid_body,
        grid=(_GRID,),
        in_specs=[_P_SPEC, _ROW_SPEC, _W_SPEC, _B_SPEC, _W_SPEC],
        out_specs=[_ROW_SPEC, _ROW_SPEC],
        out_shape=[jax.ShapeDtypeStruct((_N, _D), jnp.float32)] * 2,
    )(p, rp, wr, b.reshape(1, _D), wo)


def _mm_last(p, rp):
    return pl.pallas_call(
        _tc_last_body,
        grid=(_GRID,),
        in_specs=[_P_SPEC, _ROW_SPEC],
        out_specs=_ROW_SPEC,
        out_shape=jax.ShapeDtypeStruct((_N, _D), jnp.float32),
    )(p, rp)


def kernel(x, edge_index, edge_weight,
           W_rel_0, b_rel_0, W_root_0,
           W_rel_1, b_rel_1, W_root_1,
           W_rel_2, b_rel_2, W_root_2):
    a, r = _mm_first(x, W_rel_0, b_rel_0, W_root_0)
    p = _sc_segsum(a, edge_index, edge_weight)
    a, r = _mm_mid(p, r, W_rel_1, b_rel_1, W_root_1)
    p = _sc_segsum(a, edge_index, edge_weight)
    a, r = _mm_mid(p, r, W_rel_2, b_rel_2, W_root_2)
    p = _sc_segsum(a, edge_index, edge_weight)
    return _mm_last(p, r)
